# Initial kernel scaffold; baseline (speedup 1.0000x reference)
import jax
import jax.numpy as jnp
from jax.experimental import pallas as pl


def kernel(x, Wq, Wk, bk, Wv, bv):
    raise NotImplementedError("write your pallas kernel here")


# speedup vs baseline: 1.1035x; 1.1035x over previous
import functools

import jax
import jax.numpy as jnp
import numpy as np
from jax import lax
from jax.experimental import pallas as pl
from jax.experimental.pallas import tpu as pltpu
from jax.experimental.pallas import tpu_sc as plsc

PATCH = 4
NUM_PATCHES = 16
MAX_WINDOW = 16
BLOCK = 6
N_EMBED = 16

NUM_WORKERS = 32


def _neighbor_lists(step):
    lists = [[(step, j)] for j in range(NUM_PATCHES)]
    ii = 2
    for c in range(step, -1, -1):
        for j in range(NUM_PATCHES):
            for k in range(-ii + 1, ii):
                for l in range(-ii + 1, ii):
                    if not (j == 0 and l == 0 and ii == 2) and 0 <= j + MAX_WINDOW * k + l < NUM_PATCHES:
                        lists[j].append((c, j + MAX_WINDOW * k + l))
        ii += 1
    c_idx = np.array([c for j in range(NUM_PATCHES) for (c, p) in lists[j]], dtype=np.int64)
    p_idx = np.array([p for j in range(NUM_PATCHES) for (c, p) in lists[j]], dtype=np.int64)
    return c_idx, p_idx


def _base_of(n, c):
    hp, wp = n // 4, n % 4
    return c * 256 + hp * 64 + wp * 4


def _build_static():
    bases = []
    segments = []
    f = np.arange(BLOCK * NUM_PATCHES)
    bases.append(_base_of(f // BLOCK, f % BLOCK))
    segments.append((0, len(f), 0))
    row = len(f)
    for i in range(BLOCK):
        c_idx, p_idx = _neighbor_lists(i)
        ff = c_idx * NUM_PATCHES + p_idx
        b = _base_of(ff // (i + 1), ff % (i + 1))
        L = len(b)
        bases.append(b)
        segments.append((row, row + L, 1 + i))
        row += L
        bases.append(b)
        segments.append((row, row + L, 7 + i))
        row += L
    return np.concatenate(bases).astype(np.int32), segments, row


_BASES, _SEGMENTS, _NUM_ROWS = _build_static()
_CHUNK = -(-_NUM_ROWS // NUM_WORKERS)
_LAST_ROWS = _NUM_ROWS - (NUM_WORKERS - 1) * _CHUNK
_XPAD = BLOCK * 256 + 4 * N_EMBED
_IPAD = _NUM_ROWS + N_EMBED


def _sc_body(x_hbm, wt_hbm, b_hbm, idx_hbm, out_hbm, xv, wv, bv, idxv, buf):
    wid = lax.axis_index("s") * 2 + lax.axis_index("c")
    pltpu.sync_copy(x_hbm, xv.at[pl.ds(0, BLOCK * 256)])
    pltpu.sync_copy(wt_hbm, wv)
    pltpu.sync_copy(b_hbm, bv)
    pltpu.sync_copy(idx_hbm, idxv.at[pl.ds(0, _NUM_ROWS)])
    my_lo = wid * _CHUNK
    my_hi = jnp.minimum(my_lo + _CHUNK, _NUM_ROWS)
    for (s_lo, s_hi, w) in _SEGMENTS:
        cols = [wv[w, d] for d in range(N_EMBED)]
        bias = bv[w]
        lo = jnp.maximum(s_lo, my_lo)
        hi = jnp.maximum(lo, jnp.minimum(s_hi, my_hi))

        def body(t, carry, _cols=cols, _bias=bias, _my_lo=my_lo):
            base = idxv[pl.ds(t, N_EMBED)][0]
            acc = _bias
            for r in range(4):
                quad = xv[pl.ds(base + 16 * r, N_EMBED)]
                for j in range(4):
                    acc = acc + quad[j] * _cols[4 * r + j]
            buf[pl.ds(N_EMBED * (t - _my_lo), N_EMBED)] = acc
            return carry

        lax.fori_loop(lo, hi, body, 0)
    full = pl.ds(0, _CHUNK * N_EMBED)
    tail = pl.ds(0, _LAST_ROWS * N_EMBED)

    @pl.when(wid < NUM_WORKERS - 1)
    def _():
        pltpu.sync_copy(buf.at[full], out_hbm.at[pl.ds(my_lo * N_EMBED, _CHUNK * N_EMBED)])

    @pl.when(wid == NUM_WORKERS - 1)
    def _():
        pltpu.sync_copy(buf.at[tail], out_hbm.at[pl.ds(my_lo * N_EMBED, _LAST_ROWS * N_EMBED)])


_sc_call = pl.kernel(
    _sc_body,
    out_type=jax.ShapeDtypeStruct((_NUM_ROWS * N_EMBED,), jnp.float32),
    mesh=plsc.VectorSubcoreMesh(core_axis_name="c", subcore_axis_name="s"),
    scratch_types=[
        pltpu.VMEM((_XPAD,), jnp.float32),
        pltpu.VMEM((13, N_EMBED, N_EMBED), jnp.float32),
        pltpu.VMEM((13, N_EMBED), jnp.float32),
        pltpu.VMEM((_IPAD,), jnp.int32),
        pltpu.VMEM((_CHUNK * N_EMBED,), jnp.float32),
    ],
)


@jax.jit
def kernel(x, Wq, Wk, bk, Wv, bv):
    x_flat = x.reshape(-1)
    wcat = jnp.concatenate([Wq[None], Wk[:BLOCK], Wv[:BLOCK]], axis=0)
    wt = jnp.transpose(wcat, (0, 2, 1))
    bcat = jnp.concatenate(
        [jnp.zeros((1, N_EMBED), jnp.float32), bk[:BLOCK], bv[:BLOCK]], axis=0)
    idx = jnp.asarray(_BASES)
    return _sc_call(x_flat, wt, bcat, idx).reshape(_NUM_ROWS, N_EMBED)
